# Initial kernel scaffold; baseline (speedup 1.0000x reference)
import jax
import jax.numpy as jnp
from jax.experimental import pallas as pl


def kernel(qs, ds, rels, q_table, d_table):
    raise NotImplementedError("write your pallas kernel here")


# speedup vs baseline: 1.4273x; 1.4273x over previous
import functools

import jax
import jax.numpy as jnp
from jax import lax
from jax.experimental import pallas as pl
from jax.experimental.pallas import tpu as pltpu
from jax.experimental.pallas import tpu_sc as plsc

_NC = 2
_NS = 16
_NW = _NC * _NS
_NBUF = 4


def _build_sc_sums(B, LQ, LD, D):
    EPW = B // _NW
    NPAIR = EPW // 2
    QROWS = 2 * LQ
    D0 = (LD // 2) // 8 * 8
    D1 = LD - D0
    HALF = D // 2

    mesh = plsc.VectorSubcoreMesh(core_axis_name="c", subcore_axis_name="s")

    @functools.partial(
        pl.kernel,
        mesh=mesh,
        compiler_params=pltpu.CompilerParams(use_tc_tiling_on_sc=False),
        out_type=[
            jax.ShapeDtypeStruct((B * D,), jnp.float32),
            jax.ShapeDtypeStruct((B * D,), jnp.float32),
        ],
        scratch_types=[
            pltpu.VMEM((EPW * LQ,), jnp.int32),
            pltpu.VMEM((EPW * LD,), jnp.int32),
            pltpu.VMEM((_NBUF, QROWS, D), jnp.float32),
            pltpu.VMEM((_NBUF, LD, D), jnp.float32),
            pltpu.VMEM((EPW * D,), jnp.float32),
            pltpu.VMEM((EPW * D,), jnp.float32),
            pltpu.SemaphoreType.DMA((_NBUF,)),
            pltpu.SemaphoreType.DMA((_NBUF,)),
        ],
    )
    def sc_sums(qs_hbm, ds_hbm, qtab_hbm, dtab_hbm, oq_hbm, od_hbm,
                qidx, didx, qbufs, dbufs, qacc, dacc, qsem, dsem):
        c = lax.axis_index("c")
        s = lax.axis_index("s")
        wid = s * _NC + c
        base_e = wid * EPW

        pltpu.sync_copy(qs_hbm.at[pl.ds(base_e * LQ, EPW * LQ)], qidx)
        pltpu.sync_copy(ds_hbm.at[pl.ds(base_e * LD, EPW * LD)], didx)

        zeros = jnp.zeros((HALF,), jnp.float32)

        def q_start(p, b):
            off = pl.multiple_of(p * QROWS, 8)
            pltpu.make_async_copy(
                qtab_hbm.at[qidx.at[pl.ds(off, QROWS)]],
                qbufs.at[b], qsem.at[b]).start()

        def q_wait(p, b):
            off = pl.multiple_of(p * QROWS, 8)
            pltpu.make_async_copy(
                qtab_hbm.at[qidx.at[pl.ds(off, QROWS)]],
                qbufs.at[b], qsem.at[b]).wait()

        for b in range(_NBUF):
            q_start(b, b)

        def q_group(i, _):
            for b in range(_NBUF):
                p = i * _NBUF + b
                q_wait(p, b)
                for half_e in range(2):
                    r0 = half_e * LQ
                    lo0 = lo1 = hi0 = hi1 = zeros
                    for l in range(0, LQ, 2):
                        lo0 = lo0 + qbufs[b, r0 + l, pl.ds(0, HALF)]
                        hi0 = hi0 + qbufs[b, r0 + l, pl.ds(HALF, HALF)]
                        lo1 = lo1 + qbufs[b, r0 + l + 1, pl.ds(0, HALF)]
                        hi1 = hi1 + qbufs[b, r0 + l + 1, pl.ds(HALF, HALF)]
                    off = pl.multiple_of((2 * p + half_e) * D, 8)
                    qacc[pl.ds(off, HALF)] = lo0 + lo1
                    qacc[pl.ds(off + HALF, HALF)] = hi0 + hi1
                p_next = p + _NBUF

                @pl.when(p_next < NPAIR)
                def _():
                    q_start(p_next, b)
            return _

        lax.fori_loop(0, NPAIR // _NBUF, q_group, None)

        def d_start(e, b):
            off = pl.multiple_of(e * LD, 8)
            pltpu.make_async_copy(
                dtab_hbm.at[didx.at[pl.ds(off, D0)]],
                dbufs.at[b, pl.ds(0, D0)], dsem.at[b]).start()
            pltpu.make_async_copy(
                dtab_hbm.at[didx.at[pl.ds(off + D0, D1)]],
                dbufs.at[b, pl.ds(D0, D1)], dsem.at[b]).start()

        def d_wait(e, b):
            off = pl.multiple_of(e * LD, 8)
            pltpu.make_async_copy(
                dtab_hbm.at[didx.at[pl.ds(off, D0)]],
                dbufs.at[b, pl.ds(0, D0)], dsem.at[b]).wait()
            pltpu.make_async_copy(
                dtab_hbm.at[didx.at[pl.ds(off + D0, D1)]],
                dbufs.at[b, pl.ds(D0, D1)], dsem.at[b]).wait()

        for b in range(_NBUF):
            d_start(b, b)

        def d_group(i, _):
            for b in range(_NBUF):
                e = i * _NBUF + b
                d_wait(e, b)
                lo = [zeros] * 4
                hi = [zeros] * 4
                for l in range(0, LD, 4):
                    for u in range(4):
                        lo[u] = lo[u] + dbufs[b, l + u, pl.ds(0, HALF)]
                        hi[u] = hi[u] + dbufs[b, l + u, pl.ds(HALF, HALF)]
                off = pl.multiple_of(e * D, 8)
                dacc[pl.ds(off, HALF)] = (lo[0] + lo[1]) + (lo[2] + lo[3])
                dacc[pl.ds(off + HALF, HALF)] = (hi[0] + hi[1]) + (hi[2] + hi[3])
                e_next = e + _NBUF

                @pl.when(e_next < EPW)
                def _():
                    d_start(e_next, b)
            return _

        lax.fori_loop(0, EPW // _NBUF, d_group, None)

        pltpu.sync_copy(qacc, oq_hbm.at[pl.ds(base_e * D, EPW * D)])
        pltpu.sync_copy(dacc, od_hbm.at[pl.ds(base_e * D, EPW * D)])

    return sc_sums


def _build_combine(B, LQ, LD, D):
    def body(q_ref, d_ref, o_ref):
        q = q_ref[...] * (1.0 / LQ)
        d = d_ref[...] * (1.0 / LD)
        dot = jnp.sum(q * d, axis=1, keepdims=True)
        nq = jnp.sqrt(jnp.sum(q * q, axis=1, keepdims=True))
        nd = jnp.sqrt(jnp.sum(d * d, axis=1, keepdims=True))
        o_ref[...] = dot / (jnp.maximum(nq, 1e-12) * jnp.maximum(nd, 1e-12))

    return pl.pallas_call(
        body,
        out_shape=jax.ShapeDtypeStruct((B, 1), jnp.float32),
    )


@functools.lru_cache(maxsize=None)
def _build(B, LQ, LD, D):
    return _build_sc_sums(B, LQ, LD, D), _build_combine(B, LQ, LD, D)


def kernel(qs, ds, rels, q_table, d_table):
    B, LQ = qs.shape
    LD = ds.shape[1]
    D = q_table.shape[1]
    sc_sums, combine = _build(B, LQ, LD, D)
    q_sum, d_sum = sc_sums(qs.reshape(-1), ds.reshape(-1), q_table, d_table)
    sims = combine(q_sum.reshape(B, D), d_sum.reshape(B, D))
    return sims.reshape(B)
